# baseline (speedup 1.0000x reference)
import jax
import jax.numpy as jnp
from jax import lax
from jax.experimental import pallas as pl
from jax.experimental.pallas import tpu as pltpu
from jax.experimental.pallas import tpu_sc as plsc

N_TIME = 168
N_LAT = 360
N_LON = 720
PLANE = N_LAT * N_LON

L = 16
NW = 32
C = 992
ROW = 64
NQP = NW // 2 * ROW * C
CH_SLOW = 18
SLOW_CORE = 1

TG_PAD = 176
LG_PAD = 368


def _floor_f32(x):
    t = x.astype(jnp.int32)
    return jnp.where(t.astype(jnp.float32) > x, t - 1, t)


def _locate(x, grid_ref, scale, off, n):
    i0 = _floor_f32(x * scale + off)
    i0 = jnp.clip(i0, 0, n - 2)
    g0 = plsc.load_gather(grid_ref, [i0])
    i1 = jnp.where(g0 > x, i0 - 1, i0)
    i1 = jnp.clip(i1, 0, n - 2)
    g1 = plsc.load_gather(grid_ref, [i1 + 1])
    i2 = jnp.where(g1 <= x, i1 + 1, i1)
    i2 = jnp.clip(i2, 0, n - 2)
    ga = plsc.load_gather(grid_ref, [i2])
    gb = plsc.load_gather(grid_ref, [i2 + 1])
    w = (x - ga) / (gb - ga)
    return i2, w


def _sc_body(values_hbm, tq_hbm, la_hbm, lo_hbm, tg_hbm, lg_hbm, lon0_hbm,
             out_hbm,
             tg_v, lg_v, lon0_v,
             tqa_v, laa_v, loa_v, wa_v, idxa_v, vala_v,
             tqb_v, lab_v, lob_v, wb_v, idxb_v, valb_v,
             out_v, sema, semb, qsema, qsemb):
    bufs = (
        (tqa_v, laa_v, loa_v, wa_v, idxa_v, vala_v, sema, qsema),
        (tqb_v, lab_v, lob_v, wb_v, idxb_v, valb_v, semb, qsemb),
    )
    c = lax.axis_index("c")
    s = lax.axis_index("s")
    pltpu.sync_copy(tg_hbm, tg_v)
    pltpu.sync_copy(lg_hbm, lg_v)
    pltpu.sync_copy(lon0_hbm, lon0_v)
    lon0 = lon0_v[...]
    is_slow = c == SLOW_CORE
    nch = jnp.where(is_slow, CH_SLOW, ROW - CH_SLOW)
    off_c = jnp.where(is_slow, 0, CH_SLOW)
    base_w = (s * ROW + off_c) * C

    def fire_queries(b, ci):
        tq_v, la_v, lo_v = bufs[b][0], bufs[b][1], bufs[b][2]
        qsem = bufs[b][7]
        base = base_w + ci * C
        pltpu.async_copy(tq_hbm.at[pl.ds(base, C)], tq_v, qsem)
        pltpu.async_copy(la_hbm.at[pl.ds(base, C)], la_v, qsem)
        pltpu.async_copy(lo_hbm.at[pl.ds(base, C)], lo_v, qsem)

    def wait_queries(b, ci):
        tq_v, la_v, lo_v = bufs[b][0], bufs[b][1], bufs[b][2]
        qsem = bufs[b][7]
        base = base_w + ci * C
        pltpu.make_async_copy(tq_hbm.at[pl.ds(base, C)], tq_v, qsem).wait()
        pltpu.make_async_copy(la_hbm.at[pl.ds(base, C)], la_v, qsem).wait()
        pltpu.make_async_copy(lo_hbm.at[pl.ds(base, C)], lo_v, qsem).wait()

    def compute_fire(b):
        tq_v, la_v, lo_v, w_v, idx_v, val_v, sem, _ = bufs[b]

        def step(j, _):
            s = pl.ds(j * L, L)
            t = tq_v[s]
            la = la_v[s]
            lo = lo_v[s]
            it, wt = _locate(t, tg_v, 1.0 / 3600.0, 0.0, N_TIME)
            ila, wla = _locate(la, lg_v, 2.0, 179.5, N_LAT)
            z = lo + 180.0
            z = jnp.where(z >= 360.0, z - 360.0, z)
            bb = (z - 180.0) + 180.0
            pos = (bb - lon0) * 2.0
            pos = jnp.where(pos >= 720.0, pos - 720.0, pos)
            pos = jnp.where(pos < 0.0, pos + 720.0, pos)
            ilo = pos.astype(jnp.int32)
            wlo = pos - ilo.astype(jnp.float32)
            ilo = jnp.clip(ilo, 0, N_LON - 1)
            ilo1 = jnp.where(ilo == N_LON - 1, 0, ilo + 1)

            rowb = it * PLANE + ila * N_LON
            b0 = rowb + ilo
            b1 = rowb + ilo1
            idx_v[pl.ds(0 * C + j * L, L)] = b0
            idx_v[pl.ds(1 * C + j * L, L)] = b1
            idx_v[pl.ds(2 * C + j * L, L)] = b0 + N_LON
            idx_v[pl.ds(3 * C + j * L, L)] = b1 + N_LON
            idx_v[pl.ds(4 * C + j * L, L)] = b0 + PLANE
            idx_v[pl.ds(5 * C + j * L, L)] = b1 + PLANE
            idx_v[pl.ds(6 * C + j * L, L)] = b0 + (PLANE + N_LON)
            idx_v[pl.ds(7 * C + j * L, L)] = b1 + (PLANE + N_LON)
            w_v[0, s] = wt
            w_v[1, s] = wla
            w_v[2, s] = wlo
            return _

        lax.fori_loop(0, C // L, step, None)
        pltpu.async_copy(values_hbm.at[idx_v], val_v, sem)

    def finish(b, ci):
        w_v, idx_v, val_v, sem = bufs[b][3], bufs[b][4], bufs[b][5], bufs[b][6]
        base = base_w + ci * C
        pltpu.make_async_copy(values_hbm.at[idx_v], val_v, sem).wait()

        def comb(j, _):
            s = pl.ds(j * L, L)
            wt = w_v[0, s]
            wla = w_v[1, s]
            wlo = w_v[2, s]
            c00 = val_v[pl.ds(0 * C + j * L, L)] * (1.0 - wlo) \
                + val_v[pl.ds(1 * C + j * L, L)] * wlo
            c01 = val_v[pl.ds(2 * C + j * L, L)] * (1.0 - wlo) \
                + val_v[pl.ds(3 * C + j * L, L)] * wlo
            c10 = val_v[pl.ds(4 * C + j * L, L)] * (1.0 - wlo) \
                + val_v[pl.ds(5 * C + j * L, L)] * wlo
            c11 = val_v[pl.ds(6 * C + j * L, L)] * (1.0 - wlo) \
                + val_v[pl.ds(7 * C + j * L, L)] * wlo
            c0 = c00 * (1.0 - wla) + c01 * wla
            c1 = c10 * (1.0 - wla) + c11 * wla
            out_v[s] = c0 * (1.0 - wt) + c1 * wt
            return _

        lax.fori_loop(0, C // L, comb, None)
        pltpu.sync_copy(out_v, out_hbm.at[pl.ds(base, C)])

    fire_queries(1, 1)
    fire_queries(0, 0)
    wait_queries(0, 0)
    compute_fire(0)

    def body(k, carry):
        ci = 2 * k
        wait_queries(1, ci + 1)
        compute_fire(1)

        @pl.when(ci + 2 < nch)
        def _():
            fire_queries(0, ci + 2)

        finish(0, ci)

        @pl.when(ci + 2 < nch)
        def _():
            wait_queries(0, ci + 2)
            compute_fire(0)

        @pl.when(ci + 3 < nch)
        def _():
            fire_queries(1, ci + 3)

        finish(1, ci + 1)
        return carry

    lax.fori_loop(0, nch // 2, body, None)


@jax.jit
def _interp_sc(vflat, tq, la, lo, tg, lg, lon0):
    mesh = plsc.VectorSubcoreMesh(core_axis_name="c", subcore_axis_name="s")
    bufset = [
        pltpu.VMEM((C,), jnp.float32),
        pltpu.VMEM((C,), jnp.float32),
        pltpu.VMEM((C,), jnp.float32),
        pltpu.VMEM((3, C), jnp.float32),
        pltpu.VMEM((8 * C,), jnp.int32),
        pltpu.VMEM((8 * C,), jnp.float32),
    ]
    f = pl.kernel(
        _sc_body,
        out_type=jax.ShapeDtypeStruct((NQP,), jnp.float32),
        mesh=mesh,
        compiler_params=pltpu.CompilerParams(needs_layout_passes=False),
        scratch_types=[
            pltpu.VMEM((TG_PAD,), jnp.float32),
            pltpu.VMEM((LG_PAD,), jnp.float32),
            pltpu.VMEM((L,), jnp.float32),
        ] + bufset + bufset + [
            pltpu.VMEM((C,), jnp.float32),
            pltpu.SemaphoreType.DMA,
            pltpu.SemaphoreType.DMA,
            pltpu.SemaphoreType.DMA,
            pltpu.SemaphoreType.DMA,
        ],
    )
    return f(vflat, tq, la, lo, tg, lg, lon0)


def kernel(values, time, latitude, longitude, time_grid, lat_grid, lon_grid):
    nq = time.shape[0]
    pad = NQP - nq
    vflat = values.reshape(-1)
    tq = jnp.pad(time, (0, pad))
    la = jnp.pad(latitude, (0, pad))
    lo = jnp.pad(longitude, (0, pad))
    tg = jnp.pad(time_grid, (0, TG_PAD - N_TIME))
    lg = jnp.pad(lat_grid, (0, LG_PAD - N_LAT))
    lon0 = jnp.full((L,), lon_grid[0], dtype=jnp.float32)
    out = _interp_sc(vflat, tq, la, lo, tg, lg, lon0)
    return out[:nq]


# speedup vs baseline: 1.0379x; 1.0379x over previous
import jax
import jax.numpy as jnp
import numpy as np
from jax import lax
from jax.experimental import pallas as pl
from jax.experimental.pallas import tpu as pltpu
from jax.experimental.pallas import tpu_sc as plsc

N_TIME = 168
N_LAT = 360
N_LON = 720
PLANE = N_LAT * N_LON
LPR = N_LON // 16
LPLANE = PLANE // 16
M16 = N_TIME * LPLANE

L = 16
C = 640
ROW = 98
NQP = 16 * ROW * C
CH_SLOW = 32
SLOW_CORE = 1
FIXU = 256
NG = C // L

TG_PAD = 176
LG_PAD = 368


def _floor_f32(x):
    t = x.astype(jnp.int32)
    return jnp.where(t.astype(jnp.float32) > x, t - 1, t)


def _locate(x, grid_ref, scale, off, n):
    i0 = _floor_f32(x * scale + off)
    i0 = jnp.clip(i0, 0, n - 2)
    g0 = plsc.load_gather(grid_ref, [i0])
    i1 = jnp.where(g0 > x, i0 - 1, i0)
    i1 = jnp.clip(i1, 0, n - 2)
    g1 = plsc.load_gather(grid_ref, [i1 + 1])
    i2 = jnp.where(g1 <= x, i1 + 1, i1)
    i2 = jnp.clip(i2, 0, n - 2)
    ga = plsc.load_gather(grid_ref, [i2])
    gb = plsc.load_gather(grid_ref, [i2 + 1])
    w = (x - ga) / (gb - ga)
    return i2, w


def _sc_body(tab_hbm, tq_hbm, la_hbm, lo_hbm, tg_hbm, lg_hbm, lon0_hbm,
             out_hbm,
             tg_v, lg_v, lon0_v,
             tqa_v, laa_v, loa_v, wa_v, iloa_v, lidxa_v, rowsa_v,
             fixqa_v, fixia_v, fv0a, fv1a, fv2a, fv3a, fda_v, nfa_v,
             tqb_v, lab_v, lob_v, wb_v, ilob_v, lidxb_v, rowsb_v,
             fixqb_v, fixib_v, fv0b, fv1b, fv2b, fv3b, fdb_v, nfb_v,
             out_v, sema, semb, qsema, qsemb, fsema, fsemb):
    bufs = (
        (tqa_v, laa_v, loa_v, wa_v, iloa_v, lidxa_v, rowsa_v, fixqa_v,
         fixia_v, (fv0a, fv1a, fv2a, fv3a), fda_v, nfa_v, sema, qsema, fsema),
        (tqb_v, lab_v, lob_v, wb_v, ilob_v, lidxb_v, rowsb_v, fixqb_v,
         fixib_v, (fv0b, fv1b, fv2b, fv3b), fdb_v, nfb_v, semb, qsemb, fsemb),
    )
    c = lax.axis_index("c")
    s = lax.axis_index("s")
    _LANE = lax.iota(jnp.int32, L)
    _ZERO16 = _LANE * 0
    pltpu.sync_copy(tg_hbm, tg_v)
    pltpu.sync_copy(lg_hbm, lg_v)
    pltpu.sync_copy(lon0_hbm, lon0_v)
    lon0 = lon0_v[...]
    is_slow = c == SLOW_CORE
    nch = jnp.where(is_slow, CH_SLOW, ROW - CH_SLOW)
    off_c = jnp.where(is_slow, 0, CH_SLOW)
    base_w = (s * ROW + off_c) * C

    def init_fixidx(j, carry):
        fixia_v[pl.ds(j * L, L)] = _ZERO16
        fixib_v[pl.ds(j * L, L)] = _ZERO16
        return carry
    lax.fori_loop(0, (4 * C + 64) // L, init_fixidx, None)

    def fire_queries(b, ci):
        tq_v, la_v, lo_v = bufs[b][0], bufs[b][1], bufs[b][2]
        qsem = bufs[b][13]
        base = base_w + ci * C
        pltpu.async_copy(tq_hbm.at[pl.ds(base, C)], tq_v, qsem)
        pltpu.async_copy(la_hbm.at[pl.ds(base, C)], la_v, qsem)
        pltpu.async_copy(lo_hbm.at[pl.ds(base, C)], lo_v, qsem)

    def wait_queries(b, ci):
        tq_v, la_v, lo_v = bufs[b][0], bufs[b][1], bufs[b][2]
        qsem = bufs[b][13]
        base = base_w + ci * C
        pltpu.make_async_copy(tq_hbm.at[pl.ds(base, C)], tq_v, qsem).wait()
        pltpu.make_async_copy(la_hbm.at[pl.ds(base, C)], la_v, qsem).wait()
        pltpu.make_async_copy(lo_hbm.at[pl.ds(base, C)], lo_v, qsem).wait()

    def compute_fire(b):
        (tq_v, la_v, lo_v, w_v, ilo_v, lidx_v, rows_v, fixq_v, fixi_v,
         _fv, fd_v, nf_v, sem, _qs, fsem) = bufs[b]

        def step(j, nfix):
            sl = pl.ds(j * L, L)
            t = tq_v[sl]
            la = la_v[sl]
            lo = lo_v[sl]
            it, wt = _locate(t, tg_v, 1.0 / 3600.0, 0.0, N_TIME)
            ila, wla = _locate(la, lg_v, 2.0, 179.5, N_LAT)
            z = lo + 180.0
            z = jnp.where(z >= 360.0, z - 360.0, z)
            bb = (z - 180.0) + 180.0
            pos = (bb - lon0) * 2.0
            pos = jnp.where(pos >= 720.0, pos - 720.0, pos)
            pos = jnp.where(pos < 0.0, pos + 720.0, pos)
            ilo = pos.astype(jnp.int32)
            wlo = pos - ilo.astype(jnp.float32)
            ilo = jnp.clip(ilo, 0, N_LON - 1)

            base = it * LPLANE + ila * LPR + lax.shift_right_logical(ilo, 4)
            lidx_v[pl.ds(0 * C + j * L, L)] = base
            lidx_v[pl.ds(1 * C + j * L, L)] = base + LPR
            lidx_v[pl.ds(2 * C + j * L, L)] = base + LPLANE
            lidx_v[pl.ds(3 * C + j * L, L)] = base + (LPLANE + LPR)
            w_v[0, sl] = wt
            w_v[1, sl] = wla
            w_v[2, sl] = wlo
            ilo_v[sl] = ilo

            mask = (ilo & 15) == 15
            cnt = jnp.sum(jnp.where(mask, jnp.int32(1), jnp.int32(0)))
            qid = j * L + _LANE
            plsc.store_compressed(fixq_v.at[pl.ds(nfix, L)], qid, mask=mask)
            return nfix + cnt

        nfix = lax.fori_loop(0, NG, step, jnp.int32(0))
        nf_v[0] = nfix

        def build(g, carry):
            valid = (g * L + _LANE) < nfix
            qv = fixq_v[pl.ds(g * L, L)]
            qv = jnp.where(valid, qv, 0)
            iq = plsc.load_gather(ilo_v, [qv], mask=valid)
            delta = jnp.where(iq == N_LON - 1, jnp.int32(-(LPR - 1)),
                              jnp.int32(1))
            p4 = (g * L + _LANE) * 4
            for k in range(4):
                rv = plsc.load_gather(lidx_v, [k * C + qv], mask=valid)
                plsc.store_scatter(fixi_v, [p4 + k], rv + delta, mask=valid)
            return carry

        lax.fori_loop(0, (nfix + L - 1) // L, build, None)

        pltpu.async_copy(tab_hbm.at[lidx_v], rows_v, sem)
        pltpu.async_copy(tab_hbm.at[fixi_v.at[pl.ds(0, FIXU)]], fd_v, fsem)

    def extract_unit(b, u, nfix):
        (_tq, _la, _lo, _w, _ilo, _lidx, _rows, fixq_v, _fixi, fv, fd_v,
         _nf, _sem, _qs, _fs) = bufs[b]

        def grp(gg, carry):
            g = u * 4 + gg
            valid = (g * L + _LANE) < nfix
            qv = fixq_v[pl.ds(g * L, L)]
            qv = jnp.where(valid, qv, 0)
            p4 = (g * L + _LANE) * 4 - u * FIXU
            for k in range(4):
                vk = plsc.load_gather(fd_v, [p4 + k, _ZERO16], mask=valid)
                plsc.store_scatter(fv[k], [qv], vk, mask=valid)
            return carry

        ngrp = jnp.minimum((nfix + L - 1) // L - u * 4, 4)
        lax.fori_loop(0, ngrp, grp, None)

    def finish(b, ci):
        (_tq, _la, _lo, w_v, ilo_v, lidx_v, rows_v, _fixq, fixi_v, fv, fd_v,
         nf_v, sem, _qs, fsem) = bufs[b]
        base = base_w + ci * C
        pltpu.make_async_copy(tab_hbm.at[lidx_v], rows_v, sem).wait()
        pltpu.make_async_copy(
            tab_hbm.at[fixi_v.at[pl.ds(0, FIXU)]], fd_v, fsem).wait()
        nfix = nf_v[0]
        extract_unit(b, jnp.int32(0), nfix)

        def more_units(u, carry):
            pltpu.async_copy(tab_hbm.at[fixi_v.at[pl.ds(u * FIXU, FIXU)]],
                             fd_v, fsem).wait()
            extract_unit(b, u, nfix)
            return carry

        nunits = (4 * nfix + FIXU - 1) // FIXU
        lax.fori_loop(1, nunits, more_units, None)

        def comb(j, _):
            sl = pl.ds(j * L, L)
            wt = w_v[0, sl]
            wla = w_v[1, sl]
            wlo = w_v[2, sl]
            off = ilo_v[sl] & 15
            bnd = off == 15
            offp = jnp.where(bnd, 15, off + 1)
            qrow = j * L + _LANE
            cc = []
            for k in range(4):
                a = plsc.load_gather(rows_v, [k * C + qrow, off])
                bv = plsc.load_gather(rows_v, [k * C + qrow, offp])
                bv = jnp.where(bnd, fv[k][sl], bv)
                cc.append(a * (1.0 - wlo) + bv * wlo)
            c0 = cc[0] * (1.0 - wla) + cc[1] * wla
            c1 = cc[2] * (1.0 - wla) + cc[3] * wla
            out_v[sl] = c0 * (1.0 - wt) + c1 * wt
            return _

        lax.fori_loop(0, NG, comb, None)
        pltpu.sync_copy(out_v, out_hbm.at[pl.ds(base, C)])

    fire_queries(1, 1)
    fire_queries(0, 0)
    wait_queries(0, 0)
    compute_fire(0)

    def body(k, carry):
        ci = 2 * k
        wait_queries(1, ci + 1)
        compute_fire(1)

        @pl.when(ci + 2 < nch)
        def _():
            fire_queries(0, ci + 2)

        finish(0, ci)

        @pl.when(ci + 2 < nch)
        def _():
            wait_queries(0, ci + 2)
            compute_fire(0)

        @pl.when(ci + 3 < nch)
        def _():
            fire_queries(1, ci + 3)

        finish(1, ci + 1)
        return carry

    lax.fori_loop(0, nch // 2, body, None)


@jax.jit
def _interp_sc(tab, tq, la, lo, tg, lg, lon0):
    mesh = plsc.VectorSubcoreMesh(core_axis_name="c", subcore_axis_name="s")
    bufset = [
        pltpu.VMEM((C,), jnp.float32),
        pltpu.VMEM((C,), jnp.float32),
        pltpu.VMEM((C,), jnp.float32),
        pltpu.VMEM((3, C), jnp.float32),
        pltpu.VMEM((C,), jnp.int32),
        pltpu.VMEM((4 * C,), jnp.int32),
        pltpu.VMEM((4 * C, L), jnp.float32),
        pltpu.VMEM((C + L,), jnp.int32),
        pltpu.VMEM((4 * C + 64,), jnp.int32),
        pltpu.VMEM((C,), jnp.float32),
        pltpu.VMEM((C,), jnp.float32),
        pltpu.VMEM((C,), jnp.float32),
        pltpu.VMEM((C,), jnp.float32),
        pltpu.VMEM((FIXU, L), jnp.float32),
        pltpu.SMEM((8,), jnp.int32),
    ]
    f = pl.kernel(
        _sc_body,
        out_type=jax.ShapeDtypeStruct((NQP,), jnp.float32),
        mesh=mesh,
        compiler_params=pltpu.CompilerParams(needs_layout_passes=False,
                                             use_tc_tiling_on_sc=False),
        scratch_types=[
            pltpu.VMEM((TG_PAD,), jnp.float32),
            pltpu.VMEM((LG_PAD,), jnp.float32),
            pltpu.VMEM((L,), jnp.float32),
        ] + bufset + bufset + [
            pltpu.VMEM((C,), jnp.float32),
            pltpu.SemaphoreType.DMA,
            pltpu.SemaphoreType.DMA,
            pltpu.SemaphoreType.DMA,
            pltpu.SemaphoreType.DMA,
            pltpu.SemaphoreType.DMA,
            pltpu.SemaphoreType.DMA,
        ],
    )
    return f(tab, tq, la, lo, tg, lg, lon0)


def kernel(values, time, latitude, longitude, time_grid, lat_grid, lon_grid):
    nq = time.shape[0]
    pad = NQP - nq
    tab = values.reshape(M16, L)
    tq = jnp.pad(time, (0, pad))
    la = jnp.pad(latitude, (0, pad))
    lo = jnp.pad(longitude, (0, pad))
    tg = jnp.pad(time_grid, (0, TG_PAD - N_TIME))
    lg = jnp.pad(lat_grid, (0, LG_PAD - N_LAT))
    lon0 = jnp.full((L,), lon_grid[0], dtype=jnp.float32)
    out = _interp_sc(tab, tq, la, lo, tg, lg, lon0)
    return out[:nq]
